# baseline (speedup 1.0000x reference)
import functools

import jax
import jax.numpy as jnp
from jax import lax
from jax.experimental import pallas as pl
from jax.experimental.pallas import tpu as pltpu
from jax.experimental.pallas import tpu_sc as plsc


def _sc_gather_rows(table, idx):
    info = plsc.get_sparse_core_info()
    num_workers = info.num_cores * info.num_subcores
    (t_len,) = idx.shape
    _, d = table.shape
    rows_per_worker = t_len // num_workers
    mesh = plsc.VectorSubcoreMesh(core_axis_name="c", subcore_axis_name="s")

    @functools.partial(
        pl.kernel,
        mesh=mesh,
        out_type=jax.ShapeDtypeStruct((t_len, d), jnp.float32),
        scratch_types=[
            pltpu.VMEM((rows_per_worker,), jnp.int32),
            pltpu.VMEM((rows_per_worker, d), jnp.float32),
            pltpu.SemaphoreType.DMA,
        ],
    )
    def gather_kernel(table_hbm, idx_hbm, out_hbm, idx_v, rows_v, sem):
        wid = lax.axis_index("s") * info.num_cores + lax.axis_index("c")
        base = wid * rows_per_worker
        pltpu.sync_copy(idx_hbm.at[pl.ds(base, rows_per_worker)], idx_v)
        pltpu.async_copy(table_hbm.at[idx_v], rows_v, sem).wait()
        pltpu.sync_copy(rows_v, out_hbm.at[pl.ds(base, rows_per_worker)])

    return gather_kernel(table, idx)


def _dense_body(tidx_ref, fmul_ref, xs_ref, xi_ref, out_ref):
    i = pl.program_id(0)
    b, t_blk, d = xs_ref.shape
    xi_t = xi_ref[...]
    xs_t = xs_ref[...]
    inner = jnp.sum(xs_t * xi_t[None], axis=-1)
    sumsq = jnp.sum(xs_t * xs_t, axis=-1)
    s = jnp.where(inner > 0.0, 1.0, -1.0)
    coef = s * lax.rsqrt(jnp.sqrt(sumsq))
    m = jnp.sum(coef[:, :, None] * xs_t, axis=0) * (1.0 / b)
    msq = jnp.sum(m * m, axis=-1, keepdims=True)
    md = m * lax.rsqrt(jnp.sqrt(msq))
    iot = lax.broadcasted_iota(jnp.int32, (t_blk, 1), 0)
    fm = jnp.zeros((t_blk, 1), jnp.float32)
    for j in range(t_blk):
        fj = fmul_ref[tidx_ref[i * t_blk + j]]
        fm = fm + jnp.where(iot == j, fj, 0.0)
    out_ref[...] = s[:, :, None] * (md * fm)[None]


def kernel(xs, t, xis, f_muls):
    b, t_len, d = xs.shape
    s_len = xis.shape[0]
    tidx = jnp.round(t * (s_len - 1)).astype(jnp.int32)
    xi = _sc_gather_rows(xis, tidx)
    t_blk = 32
    return pl.pallas_call(
        _dense_body,
        grid=(t_len // t_blk,),
        in_specs=[
            pl.BlockSpec(memory_space=pltpu.SMEM),
            pl.BlockSpec(memory_space=pltpu.SMEM),
            pl.BlockSpec((b, t_blk, d), lambda i: (0, i, 0)),
            pl.BlockSpec((t_blk, d), lambda i: (i, 0)),
        ],
        out_specs=pl.BlockSpec((b, t_blk, d), lambda i: (0, i, 0)),
        out_shape=jax.ShapeDtypeStruct((b, t_len, d), jnp.float32),
        compiler_params=pltpu.CompilerParams(
            dimension_semantics=("arbitrary",),
        ),
    )(tidx, f_muls, xs, xi)
